# baseline (speedup 1.0000x reference)
import functools

import jax
import jax.numpy as jnp
from jax import lax
from jax.experimental import pallas as pl
from jax.experimental.pallas import tpu as pltpu
from jax.experimental.pallas import tpu_sc as plsc

B = 4096
EMB = 128
NNEG = 50
NC, NS = 2, 16
NW = NC * NS
UPW = B // NW
RPW = B * NNEG // NW
CHUNK = 64
NCHUNK = RPW // CHUNK
NBUF = 5
KSUP = NCHUNK // NBUF

_f32 = jnp.float32
_bf16 = jnp.bfloat16
_mesh = plsc.VectorSubcoreMesh(core_axis_name="c", subcore_axis_name="s")


def _neg_prologue(tbl_h, negidx_v, buf, g):
    for s in range(NBUF):
        pltpu.async_copy(tbl_h.at[negidx_v.at[s]], buf.at[s], g[s])


def _neg_ring(tbl_h, negidx_v, out_o, buf, g, w, rbase):
    def super_body(kk, carry):
        for s in range(NBUF):
            c = kk * NBUF + s
            row0 = rbase + c * CHUNK
            pltpu.make_async_copy(
                tbl_h.at[negidx_v.at[c]], buf.at[s], g[s]).wait()
            pltpu.async_copy(buf.at[s], out_o.at[pl.ds(row0, CHUNK)], w[s])
        for s in range(NBUF):
            c = kk * NBUF + s
            row0 = rbase + c * CHUNK
            pltpu.make_async_copy(
                buf.at[s], out_o.at[pl.ds(row0, CHUNK)], w[s]).wait()

            @pl.when(kk < KSUP - 1)
            def _():
                pltpu.async_copy(
                    tbl_h.at[negidx_v.at[c + NBUF]], buf.at[s], g[s])
        return carry

    lax.fori_loop(0, KSUP, super_body, 0)


def _sc_a(users, items, neg_idx, U_mlp, U_mf, U_mlp_g, U_mf_g, V_mlp, V_mf):
    out_type = tuple(jax.ShapeDtypeStruct((B, EMB), _f32) for _ in range(6)) \
        + (jax.ShapeDtypeStruct((B * NNEG, EMB), _f32),)

    @functools.partial(
        pl.kernel,
        mesh=_mesh,
        out_type=out_type,
        scratch_types=[
            pltpu.VMEM((UPW,), jnp.int32),
            pltpu.VMEM((NCHUNK, CHUNK), jnp.int32),
            pltpu.VMEM((2, UPW, EMB), _f32),
            pltpu.VMEM((NBUF, CHUNK, EMB), _f32),
            [pltpu.SemaphoreType.DMA] * 2,
            [pltpu.SemaphoreType.DMA] * 2,
            [pltpu.SemaphoreType.DMA] * NBUF,
            [pltpu.SemaphoreType.DMA] * NBUF,
        ],
    )
    def k(users_h, items_h, negidx_h, Umlp_h, Umf_h, Umlpg_h, Umfg_h,
          Vmlp_h, Vmf_h,
          umlp_o, umf_o, umlpg_o, umfg_o, vmlp_o, vmf_o, negmlp_o,
          idx_v, negidx_v, rows_v, buf, gs, ws, ga, wa):
        wid = lax.axis_index("s") * NC + lax.axis_index("c")
        ubase = wid * UPW
        rbase = wid * RPW
        pltpu.sync_copy(negidx_h.at[wid], negidx_v)
        _neg_prologue(Vmlp_h, negidx_v, buf, ga)
        plan = ((Umlp_h, umlp_o), (Umf_h, umf_o),
                (Umlpg_h, umlpg_o), (Umfg_h, umfg_o),
                (Vmlp_h, vmlp_o), (Vmf_h, vmf_o))
        npos = len(plan)
        pltpu.sync_copy(users_h.at[pl.ds(ubase, UPW)], idx_v)
        pltpu.async_copy(plan[0][0].at[idx_v], rows_v.at[0], gs[0])
        for n, (tbl, out) in enumerate(plan):
            s = n % 2
            pltpu.make_async_copy(tbl.at[idx_v], rows_v.at[s], gs[s]).wait()
            if n + 1 < npos:
                if n == 3:
                    pltpu.sync_copy(items_h.at[pl.ds(ubase, UPW)], idx_v)
                if n >= 1:
                    pltpu.make_async_copy(
                        rows_v.at[1 - s],
                        plan[n - 1][1].at[pl.ds(ubase, UPW)],
                        ws[1 - s]).wait()
                pltpu.async_copy(
                    plan[n + 1][0].at[idx_v], rows_v.at[1 - s], gs[1 - s])
            pltpu.async_copy(rows_v.at[s], out.at[pl.ds(ubase, UPW)], ws[s])
        for n in (npos - 2, npos - 1):
            s = n % 2
            pltpu.make_async_copy(
                rows_v.at[s], plan[n][1].at[pl.ds(ubase, UPW)], ws[s]).wait()
        _neg_ring(Vmlp_h, negidx_v, negmlp_o, buf, ga, wa, rbase)

    return k(users, items, neg_idx, U_mlp, U_mf, U_mlp_g, U_mf_g, V_mlp, V_mf)


def _sc_b(neg_idx, V_mf, order_dep):
    out_type = jax.ShapeDtypeStruct((B * NNEG, EMB), _f32)

    @functools.partial(
        pl.kernel,
        mesh=_mesh,
        out_type=out_type,
        scratch_types=[
            pltpu.VMEM((NCHUNK, CHUNK), jnp.int32),
            pltpu.VMEM((NBUF, CHUNK, EMB), _f32),
            [pltpu.SemaphoreType.DMA] * NBUF,
            [pltpu.SemaphoreType.DMA] * NBUF,
        ],
    )
    def k(negidx_h, Vmf_h, dep_h, negmf_o, negidx_v, buf, ga, wa):
        del dep_h
        wid = lax.axis_index("s") * NC + lax.axis_index("c")
        rbase = wid * RPW
        pltpu.sync_copy(negidx_h.at[wid], negidx_v)
        _neg_prologue(Vmf_h, negidx_v, buf, ga)
        _neg_ring(Vmf_h, negidx_v, negmf_o, buf, ga, wa, rbase)

    return k(neg_idx, V_mf, order_dep)


UB = 128


def _tc_mlp(u_mlp, u_mf, v_mlp, v_mf, neg_v_mlp, W1, b1, W2, b2):
    def body(umlp_r, umf_r, vmlp_r, vmf_r, nvmlp_r, W1_r, b1_r, W2_r, b2_r,
             mlp_o, mf_o, negmlp_o):
        W1u = W1_r[:EMB, :].astype(_bf16)
        W1v = W1_r[EMB:, :].astype(_bf16)
        W2 = W2_r[...].astype(_bf16)
        b1 = b1_r[...]
        b2 = b2_r[...]
        pre_u = jnp.dot(umlp_r[...].astype(_bf16), W1u,
                        preferred_element_type=_f32) + b1
        h = jnp.maximum(
            pre_u + jnp.dot(vmlp_r[...].astype(_bf16), W1v,
                            preferred_element_type=_f32), 0.0)
        mlp_o[...] = jnp.dot(h.astype(_bf16), W2,
                             preferred_element_type=_f32) + b2
        mf_o[...] = umf_r[...] * vmf_r[...]
        nv = nvmlp_r[...].reshape(NNEG * UB, EMB).astype(_bf16)
        pre_e = jnp.broadcast_to(
            pre_u[None, :, :], (NNEG, UB, EMB)).reshape(NNEG * UB, EMB)
        hn = jnp.maximum(
            pre_e + jnp.dot(nv, W1v, preferred_element_type=_f32), 0.0)
        negmlp_o[...] = (jnp.dot(hn.astype(_bf16), W2,
                                 preferred_element_type=_f32)
                         + b2).reshape(NNEG, UB, EMB)

    vec2 = pl.BlockSpec((UB, EMB), lambda i: (i, 0))
    neg3 = pl.BlockSpec((NNEG, UB, EMB), lambda i: (0, i, 0))
    full = lambda shape: pl.BlockSpec(shape, lambda i: tuple(0 for _ in shape))
    return pl.pallas_call(
        body,
        grid=(B // UB,),
        in_specs=[vec2, vec2, vec2, vec2, neg3,
                  full((2 * EMB, EMB)), full((1, EMB)),
                  full((EMB, EMB)), full((1, EMB))],
        out_specs=[vec2, vec2, neg3],
        out_shape=[
            jax.ShapeDtypeStruct((B, EMB), _f32),
            jax.ShapeDtypeStruct((B, EMB), _f32),
            jax.ShapeDtypeStruct((NNEG, B, EMB), _f32),
        ],
        compiler_params=pltpu.CompilerParams(
            dimension_semantics=("parallel",)),
    )(u_mlp, u_mf, v_mlp, v_mf, neg_v_mlp, W1, b1, W2, b2)


def _tc_mf(u_mf, neg_v_mf):
    def body(umf_r, nvmf_r, negmf_o):
        negmf_o[...] = umf_r[...][None, :, :] * nvmf_r[...]

    vec2 = pl.BlockSpec((UB, EMB), lambda i: (i, 0))
    neg3 = pl.BlockSpec((NNEG, UB, EMB), lambda i: (0, i, 0))
    return pl.pallas_call(
        body,
        grid=(B // UB,),
        in_specs=[vec2, neg3],
        out_specs=neg3,
        out_shape=jax.ShapeDtypeStruct((NNEG, B, EMB), _f32),
        compiler_params=pltpu.CompilerParams(
            dimension_semantics=("parallel",)),
    )(u_mf, neg_v_mf)


def kernel(users, items, neg_items, U_mlp, U_mf, V_mlp, V_mf, U_mlp_g, U_mf_g,
           W1, b1, W2, b2):
    users = users.astype(jnp.int32)
    items = items.astype(jnp.int32)
    neg_idx = neg_items.astype(jnp.int32).T.reshape(NW, NCHUNK, CHUNK)

    (u_mlp, u_mf, u_mlp_g, u_mf_g, v_mlp, v_mf,
     negmlp_flat) = _sc_a(
        users, items, neg_idx, U_mlp, U_mf, U_mlp_g, U_mf_g, V_mlp, V_mf)

    negmf_flat = _sc_b(neg_idx, V_mf, u_mf)

    neg_v_mlp = negmlp_flat.reshape(NNEG, B, EMB)
    neg_v_mf = negmf_flat.reshape(NNEG, B, EMB)

    mlp_vector, mf_vector, negmlp_t = _tc_mlp(
        u_mlp, u_mf, v_mlp, v_mf, neg_v_mlp,
        W1, b1.reshape(1, EMB), W2, b2.reshape(1, EMB))

    negmf_t = _tc_mf(u_mf, neg_v_mf)

    neg_mlp_vector = jnp.transpose(negmlp_t, (1, 0, 2))
    neg_mf_vector = jnp.transpose(negmf_t, (1, 0, 2))

    return (mlp_vector, mf_vector, u_mlp, u_mf, u_mlp_g, u_mf_g,
            neg_mlp_vector, neg_mf_vector)


# speedup vs baseline: 1.2538x; 1.2538x over previous
import functools

import jax
import jax.numpy as jnp
from jax import lax
from jax.experimental import pallas as pl
from jax.experimental.pallas import tpu as pltpu
from jax.experimental.pallas import tpu_sc as plsc

B = 4096
EMB = 128
NNEG = 50
NC, NS = 2, 16
NW = NC * NS
UPW = B // NW
RPW = B * NNEG // NW
CHUNK = 64
NCHUNK = RPW // CHUNK
NBUF = 5
KSUP = NCHUNK // NBUF

_f32 = jnp.float32
_bf16 = jnp.bfloat16
_mesh = plsc.VectorSubcoreMesh(core_axis_name="c", subcore_axis_name="s")


def _neg_prologue(tbl_h, negidx_v, buf, g):
    for s in range(NBUF):
        pltpu.async_copy(tbl_h.at[negidx_v.at[s]], buf.at[s], g[s])


def _neg_ring(tbl_h, negidx_v, out_o, buf, g, w, rbase):
    def super_body(kk, carry):
        for s in range(NBUF):
            c = kk * NBUF + s
            row0 = rbase + c * CHUNK
            pltpu.make_async_copy(
                tbl_h.at[negidx_v.at[c]], buf.at[s], g[s]).wait()
            pltpu.async_copy(buf.at[s], out_o.at[pl.ds(row0, CHUNK)], w[s])
        for s in range(NBUF):
            c = kk * NBUF + s
            row0 = rbase + c * CHUNK
            pltpu.make_async_copy(
                buf.at[s], out_o.at[pl.ds(row0, CHUNK)], w[s]).wait()

            @pl.when(kk < KSUP - 1)
            def _():
                pltpu.async_copy(
                    tbl_h.at[negidx_v.at[c + NBUF]], buf.at[s], g[s])
        return carry

    lax.fori_loop(0, KSUP, super_body, 0)


def _sc_a(users, items, neg_idx, U_mlp, U_mf, U_mlp_g, U_mf_g, V_mlp, V_mf):
    out_type = tuple(jax.ShapeDtypeStruct((B, EMB), _f32) for _ in range(6)) \
        + (jax.ShapeDtypeStruct((B * NNEG, EMB), _f32),)

    @functools.partial(
        pl.kernel,
        mesh=_mesh,
        out_type=out_type,
        scratch_types=[
            pltpu.VMEM((UPW,), jnp.int32),
            pltpu.VMEM((UPW,), jnp.int32),
            pltpu.VMEM((NCHUNK, CHUNK), jnp.int32),
            pltpu.VMEM((4, UPW, EMB), _f32),
            pltpu.VMEM((NBUF, CHUNK, EMB), _f32),
            [pltpu.SemaphoreType.DMA] * 4,
            [pltpu.SemaphoreType.DMA] * 4,
            [pltpu.SemaphoreType.DMA] * NBUF,
            [pltpu.SemaphoreType.DMA] * NBUF,
        ],
    )
    def k(users_h, items_h, negidx_h, Umlp_h, Umf_h, Umlpg_h, Umfg_h,
          Vmlp_h, Vmf_h,
          umlp_o, umf_o, umlpg_o, umfg_o, vmlp_o, vmf_o, negmlp_o,
          idx_v, idx2_v, negidx_v, rows_v, buf, gs, ws, ga, wa):
        wid = lax.axis_index("s") * NC + lax.axis_index("c")
        ubase = wid * UPW
        rbase = wid * RPW
        pltpu.sync_copy(negidx_h.at[wid], negidx_v)
        _neg_prologue(Vmlp_h, negidx_v, buf, ga)
        plan = ((Umlp_h, umlp_o), (Umf_h, umf_o),
                (Umlpg_h, umlpg_o), (Umfg_h, umfg_o),
                (Vmlp_h, vmlp_o), (Vmf_h, vmf_o))
        pltpu.sync_copy(users_h.at[pl.ds(ubase, UPW)], idx_v)
        for n in range(4):
            pltpu.async_copy(plan[n][0].at[idx_v], rows_v.at[n], gs[n])
        pltpu.sync_copy(items_h.at[pl.ds(ubase, UPW)], idx2_v)
        for n in range(6):
            s = n % 4
            idx = idx_v if n < 4 else idx2_v
            pltpu.make_async_copy(
                plan[n][0].at[idx], rows_v.at[s], gs[s]).wait()
            pltpu.async_copy(
                rows_v.at[s], plan[n][1].at[pl.ds(ubase, UPW)], ws[s])
            if n < 2:
                pltpu.make_async_copy(
                    rows_v.at[s], plan[n][1].at[pl.ds(ubase, UPW)],
                    ws[s]).wait()
                pltpu.async_copy(
                    plan[n + 4][0].at[idx2_v], rows_v.at[s], gs[s])
        for n in range(2, 6):
            s = n % 4
            pltpu.make_async_copy(
                rows_v.at[s], plan[n][1].at[pl.ds(ubase, UPW)], ws[s]).wait()
        _neg_ring(Vmlp_h, negidx_v, negmlp_o, buf, ga, wa, rbase)

    return k(users, items, neg_idx, U_mlp, U_mf, U_mlp_g, U_mf_g, V_mlp, V_mf)


def _sc_b(neg_idx, V_mf, u_mf_rows):
    out_type = jax.ShapeDtypeStruct((B * NNEG, EMB), _f32)

    @functools.partial(
        pl.kernel,
        mesh=_mesh,
        out_type=out_type,
        scratch_types=[
            pltpu.VMEM((NCHUNK, CHUNK), jnp.int32),
            pltpu.VMEM((NBUF, CHUNK, EMB), _f32),
            pltpu.VMEM((NBUF, CHUNK, EMB), _f32),
            pltpu.MemorySpace.VMEM_SHARED((B, EMB), _f32),
            [pltpu.SemaphoreType.DMA] * NBUF,
            [pltpu.SemaphoreType.DMA] * NBUF,
            [pltpu.SemaphoreType.DMA] * NBUF,
        ],
    )
    def k(negidx_h, Vmf_h, umf_h, negmf_o,
          negidx_v, buf, ubuf, ush, ga, ua, wa):
        sid = lax.axis_index("s")
        wid = sid * NC + lax.axis_index("c")
        rbase = wid * RPW
        pltpu.sync_copy(negidx_h.at[wid], negidx_v)
        _neg_prologue(Vmf_h, negidx_v, buf, ga)
        stage = B // NS
        pltpu.sync_copy(umf_h.at[pl.ds(sid * stage, stage)],
                        ush.at[pl.ds(sid * stage, stage)])
        plsc.subcore_barrier()

        def urow0(c):
            row0 = rbase + c * CHUNK
            return row0 - (row0 // B) * B

        for s in range(NBUF):
            pltpu.async_copy(ush.at[pl.ds(urow0(s), CHUNK)], ubuf.at[s],
                             ua[s])

        def super_body(kk, carry):
            for s in range(NBUF):
                c = kk * NBUF + s
                row0 = rbase + c * CHUNK
                pltpu.make_async_copy(
                    Vmf_h.at[negidx_v.at[c]], buf.at[s], ga[s]).wait()
                pltpu.make_async_copy(
                    ush.at[pl.ds(urow0(c), CHUNK)], ubuf.at[s], ua[s]).wait()

                def row_body(r8, carry2):
                    for rr in range(8):
                        r = r8 * 8 + rr
                        for j in range(EMB // 16):
                            sl = pl.ds(j * 16, 16)
                            buf[s, r, sl] = buf[s, r, sl] * ubuf[s, r, sl]
                    return carry2

                lax.fori_loop(0, CHUNK // 8, row_body, 0)
                pltpu.async_copy(
                    buf.at[s], negmf_o.at[pl.ds(row0, CHUNK)], wa[s])
            for s in range(NBUF):
                c = kk * NBUF + s
                row0 = rbase + c * CHUNK
                pltpu.make_async_copy(
                    buf.at[s], negmf_o.at[pl.ds(row0, CHUNK)], wa[s]).wait()

                @pl.when(kk < KSUP - 1)
                def _():
                    cn = c + NBUF
                    pltpu.async_copy(
                        Vmf_h.at[negidx_v.at[cn]], buf.at[s], ga[s])
                    pltpu.async_copy(
                        ush.at[pl.ds(urow0(cn), CHUNK)], ubuf.at[s], ua[s])
            return carry

        lax.fori_loop(0, KSUP, super_body, 0)

    return k(neg_idx, V_mf, u_mf_rows)


UB = 128


def _tc_mlp(u_mlp, u_mf, v_mlp, v_mf, neg_v_mlp, W1, b1, W2, b2):
    def body(umlp_r, umf_r, vmlp_r, vmf_r, nvmlp_r, W1_r, b1_r, W2_r, b2_r,
             mlp_o, mf_o, negmlp_o):
        W1u = W1_r[:EMB, :].astype(_bf16)
        W1v = W1_r[EMB:, :].astype(_bf16)
        W2 = W2_r[...].astype(_bf16)
        b1 = b1_r[...]
        b2 = b2_r[...]
        pre_u = jnp.dot(umlp_r[...].astype(_bf16), W1u,
                        preferred_element_type=_f32) + b1
        h = jnp.maximum(
            pre_u + jnp.dot(vmlp_r[...].astype(_bf16), W1v,
                            preferred_element_type=_f32), 0.0)
        mlp_o[...] = jnp.dot(h.astype(_bf16), W2,
                             preferred_element_type=_f32) + b2
        mf_o[...] = umf_r[...] * vmf_r[...]
        nv = nvmlp_r[...].reshape(NNEG * UB, EMB).astype(_bf16)
        pre_e = jnp.broadcast_to(
            pre_u[None, :, :], (NNEG, UB, EMB)).reshape(NNEG * UB, EMB)
        hn = jnp.maximum(
            pre_e + jnp.dot(nv, W1v, preferred_element_type=_f32), 0.0)
        negmlp_o[...] = (jnp.dot(hn.astype(_bf16), W2,
                                 preferred_element_type=_f32)
                         + b2).reshape(NNEG, UB, EMB)

    vec2 = pl.BlockSpec((UB, EMB), lambda i: (i, 0))
    neg3 = pl.BlockSpec((NNEG, UB, EMB), lambda i: (0, i, 0))
    full = lambda shape: pl.BlockSpec(shape, lambda i: tuple(0 for _ in shape))
    return pl.pallas_call(
        body,
        grid=(B // UB,),
        in_specs=[vec2, vec2, vec2, vec2, neg3,
                  full((2 * EMB, EMB)), full((1, EMB)),
                  full((EMB, EMB)), full((1, EMB))],
        out_specs=[vec2, vec2, neg3],
        out_shape=[
            jax.ShapeDtypeStruct((B, EMB), _f32),
            jax.ShapeDtypeStruct((B, EMB), _f32),
            jax.ShapeDtypeStruct((NNEG, B, EMB), _f32),
        ],
        compiler_params=pltpu.CompilerParams(
            dimension_semantics=("parallel",)),
    )(u_mlp, u_mf, v_mlp, v_mf, neg_v_mlp, W1, b1, W2, b2)


def kernel(users, items, neg_items, U_mlp, U_mf, V_mlp, V_mf, U_mlp_g, U_mf_g,
           W1, b1, W2, b2):
    users = users.astype(jnp.int32)
    items = items.astype(jnp.int32)
    neg_idx = neg_items.astype(jnp.int32).T.reshape(NW, NCHUNK, CHUNK)

    (u_mlp, u_mf, u_mlp_g, u_mf_g, v_mlp, v_mf,
     negmlp_flat) = _sc_a(
        users, items, neg_idx, U_mlp, U_mf, U_mlp_g, U_mf_g, V_mlp, V_mf)

    negmf_flat = _sc_b(neg_idx, V_mf, u_mf)

    neg_v_mlp = negmlp_flat.reshape(NNEG, B, EMB)
    negmf_t = negmf_flat.reshape(NNEG, B, EMB)

    mlp_vector, mf_vector, negmlp_t = _tc_mlp(
        u_mlp, u_mf, v_mlp, v_mf, neg_v_mlp,
        W1, b1.reshape(1, EMB), W2, b2.reshape(1, EMB))

    neg_mlp_vector = jnp.transpose(negmlp_t, (1, 0, 2))
    neg_mf_vector = jnp.transpose(negmf_t, (1, 0, 2))

    return (mlp_vector, mf_vector, u_mlp, u_mf, u_mlp_g, u_mf_g,
            neg_mlp_vector, neg_mf_vector)
